# baseline (speedup 1.0000x reference)
import functools

import jax
import jax.numpy as jnp
from jax import lax
from jax.experimental import pallas as pl
from jax.experimental.pallas import tpu as pltpu
from jax.experimental.pallas import tpu_sc as plsc

NUM_SEG = 4096
N_ATOMS = 800000
HID = 64

_GRIDT = 250
_SUB = 25
_K = 5
_CBLK = _GRIDT // _K
_CATOMS = N_ATOMS // _K


def _matvec_body(x_ref, w_ref, o_ref):
    x = x_ref[...]
    w = w_ref[...]
    prod = x * w[None, None, :, :]
    o_ref[...] = jnp.sum(prod, axis=3)


def _matvec_chunk(x4, w_row, k):
    return pl.pallas_call(
        _matvec_body,
        grid=(_CBLK,),
        in_specs=[
            pl.BlockSpec((1, _SUB, 128, HID),
                         lambda i, k=k: (k * _CBLK + i, 0, 0, 0)),
            pl.BlockSpec((1, HID), lambda i: (0, 0)),
        ],
        out_specs=pl.BlockSpec((1, _SUB, 128), lambda i: (i, 0, 0)),
        out_shape=jax.ShapeDtypeStruct((_CBLK, _SUB, 128), jnp.float32),
    )(x4, w_row)


_NW = 32
_CHUNK = _CATOMS // _NW
_VECS = _CHUNK // 16
_TAIL = _CHUNK - _VECS * 16
_CPAD = _CHUNK + (16 - _TAIL) % 16


def _make_segsum_body(k):
    def _segsum_body(r_hbm, ids_hbm, out_hbm, rv, gv, acc):
        nc = plsc.get_sparse_core_info().num_cores
        wid = lax.axis_index("s") * nc + lax.axis_index("c")
        base = wid * _CHUNK
        gbase = k * _CATOMS + wid * _CHUNK
        pltpu.sync_copy(r_hbm.at[pl.ds(base, _CHUNK)], rv.at[pl.ds(0, _CHUNK)])
        pltpu.sync_copy(ids_hbm.at[pl.ds(gbase, _CHUNK)],
                        gv.at[pl.ds(0, _CHUNK)])

        def zero_body(j, _):
            for u in range(8):
                acc[j, pl.ds(u * 16, 16)] = jnp.zeros((16,), jnp.float32)
            return 0

        lax.fori_loop(0, NUM_SEG // 128, zero_body, 0)

        def body(i, _):
            v = rv[pl.ds(i * 16, 16)]
            g = gv[pl.ds(i * 16, 16)]
            plsc.addupdate_scatter(acc, [g >> 7, g & 127], v)
            return 0

        lax.fori_loop(0, _VECS, body, 0)
        if _TAIL:
            v = rv[pl.ds(_VECS * 16, 16)]
            g = gv[pl.ds(_VECS * 16, 16)]
            m = lax.iota(jnp.int32, 16) < _TAIL
            plsc.addupdate_scatter(acc, [g >> 7, g & 127], v, mask=m)
        pltpu.sync_copy(acc, out_hbm.at[wid // 8, :, wid % 8, :])

    return _segsum_body


def _segsum_chunk(r, ids, k):
    f = functools.partial(
        pl.kernel,
        out_type=jax.ShapeDtypeStruct((_NW // 8, NUM_SEG // 128, 8, 128),
                                      jnp.float32),
        mesh=plsc.VectorSubcoreMesh(core_axis_name="c", subcore_axis_name="s"),
        scratch_types=[
            pltpu.VMEM((_CPAD,), jnp.float32),
            pltpu.VMEM((_CPAD,), jnp.int32),
            pltpu.VMEM((NUM_SEG // 128, 128), jnp.float32),
        ],
        compiler_params=pltpu.CompilerParams(needs_layout_passes=False),
    )(_make_segsum_body(k))
    return f(r, ids)


def _combine_body(*refs):
    p_refs, b_ref, o_ref = refs[:_K], refs[_K], refs[_K + 1]
    p = p_refs[0][...]
    for pr in p_refs[1:]:
        p = p + pr[...]
    s = jnp.sum(p, axis=(0, 2))
    o_ref[...] = s + b_ref[0]


def _combine(partials, b):
    return pl.pallas_call(
        _combine_body,
        in_specs=[pl.BlockSpec(memory_space=pltpu.VMEM)] * _K
        + [pl.BlockSpec(memory_space=pltpu.SMEM)],
        out_specs=pl.BlockSpec(memory_space=pltpu.VMEM),
        out_shape=jax.ShapeDtypeStruct((NUM_SEG // 128, 128), jnp.float32),
    )(*partials, b)


def kernel(atom_feats, segment_ids, W, b):
    x4 = atom_feats.reshape(_GRIDT, _SUB, 128, HID)
    w_row = W.reshape(1, HID)
    ids = segment_ids.astype(jnp.int32)
    partials = []
    for k in range(_K):
        r_k = _matvec_chunk(x4, w_row, k).reshape(_CATOMS)
        partials.append(_segsum_chunk(r_k, ids, k))
    return _combine(partials, b).reshape(NUM_SEG, 1)


# speedup vs baseline: 1.3071x; 1.3071x over previous
import functools

import jax
import jax.numpy as jnp
from jax import lax
from jax.experimental import pallas as pl
from jax.experimental.pallas import tpu as pltpu
from jax.experimental.pallas import tpu_sc as plsc

NUM_SEG = 4096
N_ATOMS = 800000
HID = 64

_GRIDT = 50
_SUB = 125
_K = 5
_CBLK = _GRIDT // _K
_CATOMS = N_ATOMS // _K


def _matvec_body(x_ref, w_ref, o_ref):
    x = x_ref[...]
    w = w_ref[...]
    prod = x * w[None, None, :, :]
    o_ref[...] = jnp.sum(prod, axis=3)


def _matvec_chunk(x4, w_row, k):
    return pl.pallas_call(
        _matvec_body,
        grid=(_CBLK,),
        in_specs=[
            pl.BlockSpec((1, _SUB, 128, HID),
                         lambda i, k=k: (k * _CBLK + i, 0, 0, 0)),
            pl.BlockSpec((1, HID), lambda i: (0, 0)),
        ],
        out_specs=pl.BlockSpec((1, _SUB, 128), lambda i: (i, 0, 0)),
        out_shape=jax.ShapeDtypeStruct((_CBLK, _SUB, 128), jnp.float32),
    )(x4, w_row)


_NW = 32
_CHUNK = _CATOMS // _NW
_VECS = _CHUNK // 16
_TAIL = _CHUNK - _VECS * 16
_CPAD = _CHUNK + (16 - _TAIL) % 16


def _make_segsum_body(k):
    def _segsum_body(r_hbm, ids_hbm, out_hbm, rv, gv, acc):
        nc = plsc.get_sparse_core_info().num_cores
        wid = lax.axis_index("s") * nc + lax.axis_index("c")
        base = wid * _CHUNK
        gbase = k * _CATOMS + wid * _CHUNK
        pltpu.sync_copy(r_hbm.at[pl.ds(base, _CHUNK)], rv.at[pl.ds(0, _CHUNK)])
        pltpu.sync_copy(ids_hbm.at[pl.ds(gbase, _CHUNK)],
                        gv.at[pl.ds(0, _CHUNK)])

        def zero_body(j, _):
            for u in range(8):
                acc[j, pl.ds(u * 16, 16)] = jnp.zeros((16,), jnp.float32)
            return 0

        lax.fori_loop(0, NUM_SEG // 128, zero_body, 0)

        def body(i, _):
            v = rv[pl.ds(i * 16, 16)]
            g = gv[pl.ds(i * 16, 16)]
            plsc.addupdate_scatter(acc, [g >> 7, g & 127], v)
            return 0

        lax.fori_loop(0, _VECS, body, 0)
        if _TAIL:
            v = rv[pl.ds(_VECS * 16, 16)]
            g = gv[pl.ds(_VECS * 16, 16)]
            m = lax.iota(jnp.int32, 16) < _TAIL
            plsc.addupdate_scatter(acc, [g >> 7, g & 127], v, mask=m)
        pltpu.sync_copy(acc, out_hbm.at[wid // 8, :, wid % 8, :])

    return _segsum_body


def _segsum_chunk(r, ids, k):
    f = functools.partial(
        pl.kernel,
        out_type=jax.ShapeDtypeStruct((_NW // 8, NUM_SEG // 128, 8, 128),
                                      jnp.float32),
        mesh=plsc.VectorSubcoreMesh(core_axis_name="c", subcore_axis_name="s"),
        scratch_types=[
            pltpu.VMEM((_CPAD,), jnp.float32),
            pltpu.VMEM((_CPAD,), jnp.int32),
            pltpu.VMEM((NUM_SEG // 128, 128), jnp.float32),
        ],
        compiler_params=pltpu.CompilerParams(needs_layout_passes=False),
    )(_make_segsum_body(k))
    return f(r, ids)


def _combine_body(*refs):
    p_refs, b_ref, o_ref = refs[:_K], refs[_K], refs[_K + 1]
    p = p_refs[0][...]
    for pr in p_refs[1:]:
        p = p + pr[...]
    s = jnp.sum(p, axis=(0, 2))
    o_ref[...] = s + b_ref[0]


def _combine(partials, b):
    return pl.pallas_call(
        _combine_body,
        in_specs=[pl.BlockSpec(memory_space=pltpu.VMEM)] * _K
        + [pl.BlockSpec(memory_space=pltpu.SMEM)],
        out_specs=pl.BlockSpec(memory_space=pltpu.VMEM),
        out_shape=jax.ShapeDtypeStruct((NUM_SEG // 128, 128), jnp.float32),
    )(*partials, b)


def kernel(atom_feats, segment_ids, W, b):
    x4 = atom_feats.reshape(_GRIDT, _SUB, 128, HID)
    w_row = W.reshape(1, HID)
    ids = segment_ids.astype(jnp.int32)
    partials = []
    for k in range(_K):
        r_k = _matvec_chunk(x4, w_row, k).reshape(_CATOMS)
        partials.append(_segsum_chunk(r_k, ids, k))
    return _combine(partials, b).reshape(NUM_SEG, 1)
